# Initial kernel scaffold; baseline (speedup 1.0000x reference)
import functools
import jax
import jax.numpy as jnp
from jax import lax
from jax.experimental import pallas as pl
from jax.experimental.pallas import tpu as pltpu
from jax.experimental.pallas import tpu_sc as plsc

N = 10000
D = 128
E = 320000
NC = 2
NS = 16
NW = NC * NS
EPW = E // NW
CHUNK = 80
NCHUNK = EPW // CHUNK
GROUPS = CHUNK // 16
RPT = N // NS


def _bcast_lane(v16, lane):
    idx = jnp.full((16, 1), lane, dtype=jnp.int32)
    return lax.gather(
        v16, idx,
        dimension_numbers=lax.GatherDimensionNumbers(
            offset_dims=(), collapsed_slice_dims=(0,), start_index_map=(0,)),
        slice_sizes=(1,),
        mode=lax.GatherScatterMode.PROMISE_IN_BOUNDS)


def _sc_body(g_hbm, col_hbm, row_hbm, val_hbm, zero_hbm, p_hbm,
             col_v, row_v, val_v, rows_v, acc_sh, sem):
    c = lax.axis_index("c")
    s = lax.axis_index("s")
    wid = s * NC + c
    base = wid * EPW

    pltpu.sync_copy(zero_hbm, acc_sh.at[pl.ds(s * RPT, RPT)])
    plsc.subcore_barrier()

    def chunk(j, carry):
        off = base + j * CHUNK
        pltpu.sync_copy(col_hbm.at[pl.ds(off, CHUNK)], col_v)
        pltpu.sync_copy(row_hbm.at[pl.ds(off, CHUNK)], row_v)
        pltpu.sync_copy(val_hbm.at[pl.ds(off, CHUNK)], val_v)
        pltpu.async_copy(g_hbm.at[col_v], rows_v, sem).wait()
        for g in range(GROUPS):
            v16 = val_v[pl.ds(g * 16, 16)]
            for i in range(16):
                e = g * 16 + i
                sc = _bcast_lane(v16, i)
                for d in range(D // 16):
                    sl = pl.ds(d * 16, 16)
                    rows_v[e, sl] = rows_v[e, sl] * sc
        pltpu.sync_copy(rows_v, acc_sh.at[row_v], add=True)
        return carry

    lax.fori_loop(0, NCHUNK, chunk, 0)

    plsc.subcore_barrier()
    pltpu.sync_copy(acc_sh.at[pl.ds(s * RPT, RPT)],
                    p_hbm.at[pl.ds(c * N + s * RPT, RPT)])


@functools.partial(
    pl.kernel,
    out_type=jax.ShapeDtypeStruct((2 * N, D), jnp.float32),
    mesh=plsc.VectorSubcoreMesh(core_axis_name="c", subcore_axis_name="s"),
    scratch_types=[
        pltpu.VMEM((CHUNK,), jnp.int32),
        pltpu.VMEM((CHUNK,), jnp.int32),
        pltpu.VMEM((CHUNK,), jnp.float32),
        pltpu.VMEM((CHUNK, D), jnp.float32),
        pltpu.VMEM_SHARED((N, D), jnp.float32),
        pltpu.SemaphoreType.DMA,
    ],
)
def _sc_spmm(g_hbm, col_hbm, row_hbm, val_hbm, zero_hbm, p_hbm,
             col_v, row_v, val_v, rows_v, acc_sh, sem):
    _sc_body(g_hbm, col_hbm, row_hbm, val_hbm, zero_hbm, p_hbm,
             col_v, row_v, val_v, rows_v, acc_sh, sem)


_BLK = 2000


def _mm_a_body(h_ref, w_ref, o_ref):
    o_ref[...] = jnp.dot(h_ref[...], w_ref[...],
                         preferred_element_type=jnp.float32)


def _mm_b_body(h_ref, w_ref, b_ref, p0_ref, p1_ref, o_ref):
    o_ref[...] = (jnp.dot(h_ref[...], w_ref[...],
                          preferred_element_type=jnp.float32)
                  + b_ref[...] + p0_ref[...] + p1_ref[...])


def kernel(H, A_indices, A_values, W, b):
    col = A_indices[1].astype(jnp.int32)
    row = A_indices[0].astype(jnp.int32)
    w1t = W[:, :D].T
    w2t = W[:, D:].T
    zeros = jnp.zeros((RPT, D), jnp.float32)
    b2 = b.reshape(1, D)

    G = pl.pallas_call(
        _mm_a_body,
        grid=(N // _BLK,),
        in_specs=[
            pl.BlockSpec((_BLK, D), lambda i: (i, 0)),
            pl.BlockSpec((D, D), lambda i: (0, 0)),
        ],
        out_specs=pl.BlockSpec((_BLK, D), lambda i: (i, 0)),
        out_shape=jax.ShapeDtypeStruct((N, D), jnp.float32),
    )(H, w2t)

    P = _sc_spmm(G, col, row, A_values, zeros)

    Y = pl.pallas_call(
        _mm_b_body,
        grid=(N // _BLK,),
        in_specs=[
            pl.BlockSpec((_BLK, D), lambda i: (i, 0)),
            pl.BlockSpec((D, D), lambda i: (0, 0)),
            pl.BlockSpec((1, D), lambda i: (0, 0)),
            pl.BlockSpec((_BLK, D), lambda i: (i, 0)),
            pl.BlockSpec((_BLK, D), lambda i: (i + N // _BLK, 0)),
        ],
        out_specs=pl.BlockSpec((_BLK, D), lambda i: (i, 0)),
        out_shape=jax.ShapeDtypeStruct((N, D), jnp.float32),
    )(H, w1t, b2, P, P)

    return Y


# speedup vs baseline: 4.4375x; 4.4375x over previous
import functools
import jax
import jax.numpy as jnp
from jax import lax
from jax.experimental import pallas as pl
from jax.experimental.pallas import tpu as pltpu
from jax.experimental.pallas import tpu_sc as plsc

N = 10000
D = 128
E = 320000
NC = 2
NS = 16
NW = NC * NS
EPW = E // NW
CHUNK = 80
NCHUNK = EPW // CHUNK
GROUPS = CHUNK // 16
RPT = 624
TAIL = N - NS * RPT


def _bcast_lane(v16, lane):
    idx = jnp.full((16, 1), lane, dtype=jnp.int32)
    return lax.gather(
        v16, idx,
        dimension_numbers=lax.GatherDimensionNumbers(
            offset_dims=(), collapsed_slice_dims=(0,), start_index_map=(0,)),
        slice_sizes=(1,),
        mode=lax.GatherScatterMode.PROMISE_IN_BOUNDS)


def _sc_body(g_hbm, col_hbm, row_hbm, val_hbm, zero_hbm, p_hbm,
             col_v, row_v, val_v, rows_v, acc_sh, sem):
    c = lax.axis_index("c")
    s = lax.axis_index("s")
    wid = s * NC + c
    base = wid * EPW

    pltpu.sync_copy(zero_hbm.at[pl.ds(0, RPT)],
                    acc_sh.at[pl.ds(s * RPT, RPT)])

    @pl.when(s == 0)
    def _():
        pltpu.sync_copy(zero_hbm.at[pl.ds(0, TAIL)],
                        acc_sh.at[pl.ds(NS * RPT, TAIL)])

    plsc.subcore_barrier()

    def chunk(j, carry):
        off = base + j * CHUNK
        pltpu.sync_copy(col_hbm.at[pl.ds(off, CHUNK)], col_v)
        pltpu.sync_copy(row_hbm.at[pl.ds(off, CHUNK)], row_v)
        pltpu.sync_copy(val_hbm.at[pl.ds(off, CHUNK)], val_v)
        pltpu.async_copy(g_hbm.at[col_v], rows_v, sem).wait()
        for g in range(GROUPS):
            v16 = val_v[pl.ds(g * 16, 16)]
            for i in range(16):
                e = g * 16 + i
                sc = _bcast_lane(v16, i)
                for d in range(D // 16):
                    sl = pl.ds(d * 16, 16)
                    rows_v[e, sl] = rows_v[e, sl] * sc
        pltpu.sync_copy(rows_v, acc_sh.at[row_v], add=True)
        return carry

    lax.fori_loop(0, NCHUNK, chunk, 0)

    plsc.subcore_barrier()
    pltpu.sync_copy(acc_sh.at[pl.ds(s * RPT, RPT)],
                    p_hbm.at[pl.ds(c * N + s * RPT, RPT)])

    @pl.when(s == 0)
    def _():
        pltpu.sync_copy(acc_sh.at[pl.ds(NS * RPT, TAIL)],
                        p_hbm.at[pl.ds(c * N + NS * RPT, TAIL)])


@functools.partial(
    pl.kernel,
    out_type=jax.ShapeDtypeStruct((2 * N, D), jnp.float32),
    mesh=plsc.VectorSubcoreMesh(core_axis_name="c", subcore_axis_name="s",
                                num_cores=NC, num_subcores=NS),
    scratch_types=[
        pltpu.VMEM((CHUNK,), jnp.int32),
        pltpu.VMEM((CHUNK,), jnp.int32),
        pltpu.VMEM((CHUNK,), jnp.float32),
        pltpu.VMEM((CHUNK, D), jnp.float32),
        pltpu.VMEM_SHARED((N, D), jnp.float32),
        pltpu.SemaphoreType.DMA,
    ],
)
def _sc_spmm(g_hbm, col_hbm, row_hbm, val_hbm, zero_hbm, p_hbm,
             col_v, row_v, val_v, rows_v, acc_sh, sem):
    _sc_body(g_hbm, col_hbm, row_hbm, val_hbm, zero_hbm, p_hbm,
             col_v, row_v, val_v, rows_v, acc_sh, sem)


_BLK = 2000


def _mm_a_body(h_ref, w_ref, o_ref):
    o_ref[...] = jnp.dot(h_ref[...], w_ref[...],
                         preferred_element_type=jnp.float32)


def _mm_b_body(h_ref, w_ref, b_ref, p0_ref, p1_ref, o_ref):
    o_ref[...] = (jnp.dot(h_ref[...], w_ref[...],
                          preferred_element_type=jnp.float32)
                  + b_ref[...] + p0_ref[...] + p1_ref[...])


def kernel(H, A_indices, A_values, W, b):
    col = A_indices[1].astype(jnp.int32)
    row = A_indices[0].astype(jnp.int32)
    w1t = W[:, :D].T
    w2t = W[:, D:].T
    zeros = jnp.zeros((RPT, D), jnp.float32)
    b2 = b.reshape(1, D)

    G = pl.pallas_call(
        _mm_a_body,
        grid=(N // _BLK,),
        in_specs=[
            pl.BlockSpec((_BLK, D), lambda i: (i, 0)),
            pl.BlockSpec((D, D), lambda i: (0, 0)),
        ],
        out_specs=pl.BlockSpec((_BLK, D), lambda i: (i, 0)),
        out_shape=jax.ShapeDtypeStruct((N, D), jnp.float32),
    )(H, w2t)

    P = _sc_spmm(G, col, row, A_values, zeros)

    Y = pl.pallas_call(
        _mm_b_body,
        grid=(N // _BLK,),
        in_specs=[
            pl.BlockSpec((_BLK, D), lambda i: (i, 0)),
            pl.BlockSpec((D, D), lambda i: (0, 0)),
            pl.BlockSpec((1, D), lambda i: (0, 0)),
            pl.BlockSpec((_BLK, D), lambda i: (i, 0)),
            pl.BlockSpec((_BLK, D), lambda i: (i + N // _BLK, 0)),
        ],
        out_specs=pl.BlockSpec((_BLK, D), lambda i: (i, 0)),
        out_shape=jax.ShapeDtypeStruct((N, D), jnp.float32),
    )(H, w1t, b2, P, P)

    return Y


# speedup vs baseline: 9.6970x; 2.1852x over previous
import functools
import jax
import jax.numpy as jnp
from jax import lax
from jax.experimental import pallas as pl
from jax.experimental.pallas import tpu as pltpu
from jax.experimental.pallas import tpu_sc as plsc

N = 10000
D = 128
E = 320000
NC = 2
NS = 16
NW = NC * NS
EPW = E // NW
CHUNK = 80
NCHUNK = EPW // CHUNK
GROUPS = CHUNK // 16
RPT = 624
TAIL = N - NS * RPT


def _bcast_lane(v16, lane):
    idx = jnp.full((16, 1), lane, dtype=jnp.int32)
    return lax.gather(
        v16, idx,
        dimension_numbers=lax.GatherDimensionNumbers(
            offset_dims=(), collapsed_slice_dims=(0,), start_index_map=(0,)),
        slice_sizes=(1,),
        mode=lax.GatherScatterMode.PROMISE_IN_BOUNDS)


def _sc_body(g_hbm, col_hbm, row_hbm, val_hbm, zero_hbm, p_hbm,
             col_v, row_sl, val_sl, rows_a, rows_b, acc_sh,
             sem_a, sem_b, sem_i):
    c = lax.axis_index("c")
    s = lax.axis_index("s")
    wid = s * NC + c
    base = wid * EPW

    pltpu.sync_copy(zero_hbm.at[pl.ds(0, RPT)],
                    acc_sh.at[pl.ds(s * RPT, RPT)])

    @pl.when(s == 0)
    def _():
        pltpu.sync_copy(zero_hbm.at[pl.ds(0, TAIL)],
                        acc_sh.at[pl.ds(NS * RPT, TAIL)])

    pltpu.sync_copy(col_hbm.at[wid], col_v)
    plsc.subcore_barrier()

    def gather(j, buf, sem):
        pltpu.async_copy(g_hbm.at[col_v.at[j]], buf, sem)

    def stage_idx(j, p):
        off = pl.ds(base + j * CHUNK, CHUNK)
        pltpu.async_copy(row_hbm.at[off], row_sl.at[p], sem_i)
        pltpu.async_copy(val_hbm.at[off], val_sl.at[p], sem_i)

    def wait_idx(j, p):
        off = pl.ds(base + j * CHUNK, CHUNK)
        pltpu.make_async_copy(row_hbm.at[off], row_sl.at[p], sem_i).wait()
        pltpu.make_async_copy(val_hbm.at[off], val_sl.at[p], sem_i).wait()

    def process(j, buf, p):
        wait_idx(j, p)
        for g in range(GROUPS):
            v16 = val_sl[p, pl.ds(g * 16, 16)]
            for i in range(16):
                e = g * 16 + i
                sc = _bcast_lane(v16, i)
                for d in range(D // 16):
                    sl = pl.ds(d * 16, 16)
                    buf[e, sl] = buf[e, sl] * sc
        pltpu.sync_copy(buf, acc_sh.at[row_sl.at[p]], add=True)

    stage_idx(0, 0)
    gather(0, rows_a, sem_a)

    def pair(k, carry):
        j = 2 * k
        pltpu.make_async_copy(g_hbm.at[col_v.at[j]], rows_a, sem_a).wait()
        gather(j + 1, rows_b, sem_b)
        stage_idx(j + 1, 1)
        process(j, rows_a, 0)
        pltpu.make_async_copy(g_hbm.at[col_v.at[j + 1]], rows_b, sem_b).wait()
        gather(j + 2, rows_a, sem_a)
        stage_idx(j + 2, 0)
        process(j + 1, rows_b, 1)
        return carry

    lax.fori_loop(0, (NCHUNK - 1) // 2, pair, 0)

    pltpu.make_async_copy(g_hbm.at[col_v.at[NCHUNK - 1]], rows_a, sem_a).wait()
    process(NCHUNK - 1, rows_a, 0)

    plsc.subcore_barrier()
    pltpu.sync_copy(acc_sh.at[pl.ds(s * RPT, RPT)],
                    p_hbm.at[pl.ds(c * N + s * RPT, RPT)])

    @pl.when(s == 0)
    def _():
        pltpu.sync_copy(acc_sh.at[pl.ds(NS * RPT, TAIL)],
                        p_hbm.at[pl.ds(c * N + NS * RPT, TAIL)])


@functools.partial(
    pl.kernel,
    out_type=jax.ShapeDtypeStruct((2 * N, D), jnp.float32),
    mesh=plsc.VectorSubcoreMesh(core_axis_name="c", subcore_axis_name="s",
                                num_cores=NC, num_subcores=NS),
    scratch_types=[
        pltpu.VMEM((NCHUNK, CHUNK), jnp.int32),
        pltpu.VMEM((2, CHUNK), jnp.int32),
        pltpu.VMEM((2, CHUNK), jnp.float32),
        pltpu.VMEM((CHUNK, D), jnp.float32),
        pltpu.VMEM((CHUNK, D), jnp.float32),
        pltpu.VMEM_SHARED((N, D), jnp.float32),
        pltpu.SemaphoreType.DMA,
        pltpu.SemaphoreType.DMA,
        pltpu.SemaphoreType.DMA,
    ],
)
def _sc_spmm(g_hbm, col_hbm, row_hbm, val_hbm, zero_hbm, p_hbm,
             col_v, row_sl, val_sl, rows_a, rows_b, acc_sh,
             sem_a, sem_b, sem_i):
    _sc_body(g_hbm, col_hbm, row_hbm, val_hbm, zero_hbm, p_hbm,
             col_v, row_sl, val_sl, rows_a, rows_b, acc_sh,
             sem_a, sem_b, sem_i)


_BLK = 2000


def _mm_a_body(h_ref, w_ref, o_ref):
    o_ref[...] = jnp.dot(h_ref[...], w_ref[...],
                         preferred_element_type=jnp.float32)


def _mm_b_body(h_ref, w_ref, b_ref, p0_ref, p1_ref, o_ref):
    o_ref[...] = (jnp.dot(h_ref[...], w_ref[...],
                          preferred_element_type=jnp.float32)
                  + b_ref[...] + p0_ref[...] + p1_ref[...])


def kernel(H, A_indices, A_values, W, b):
    col = A_indices[1].astype(jnp.int32).reshape(NW, NCHUNK, CHUNK)
    row = A_indices[0].astype(jnp.int32)
    w1t = W[:, :D].T
    w2t = W[:, D:].T
    zeros = jnp.zeros((RPT, D), jnp.float32)
    b2 = b.reshape(1, D)

    G = pl.pallas_call(
        _mm_a_body,
        grid=(N // _BLK,),
        in_specs=[
            pl.BlockSpec((_BLK, D), lambda i: (i, 0)),
            pl.BlockSpec((D, D), lambda i: (0, 0)),
        ],
        out_specs=pl.BlockSpec((_BLK, D), lambda i: (i, 0)),
        out_shape=jax.ShapeDtypeStruct((N, D), jnp.float32),
    )(H, w2t)

    P = _sc_spmm(G, col, row, A_values, zeros)

    Y = pl.pallas_call(
        _mm_b_body,
        grid=(N // _BLK,),
        in_specs=[
            pl.BlockSpec((_BLK, D), lambda i: (i, 0)),
            pl.BlockSpec((D, D), lambda i: (0, 0)),
            pl.BlockSpec((1, D), lambda i: (0, 0)),
            pl.BlockSpec((_BLK, D), lambda i: (i, 0)),
            pl.BlockSpec((_BLK, D), lambda i: (i + N // _BLK, 0)),
        ],
        out_specs=pl.BlockSpec((_BLK, D), lambda i: (i, 0)),
        out_shape=jax.ShapeDtypeStruct((N, D), jnp.float32),
    )(H, w1t, b2, P, P)

    return Y
